# baseline (speedup 1.0000x reference)
import functools
import math

import jax
import jax.numpy as jnp
from jax import lax
from jax.experimental import pallas as pl
from jax.experimental.pallas import tpu as pltpu
from jax.experimental.pallas import tpu_sc as plsc

MODEL_DIM = 64
SCALE = math.sqrt(MODEL_DIM)
LANES = 16
NUM_WORKERS = 32
BB = 128
NG = 4
NS = 2


def _emb_kernel_body(seq, idxt_hbm, w_hbm, out_hbm, idx_v, rg0, rg1, rg2, rg3,
                     ov0, ov1, gs0, gs1, gs2, gs3, ss0, ss1):
    rgs = (rg0, rg1, rg2, rg3)
    ovs = (ov0, ov1)
    gsems = (gs0, gs1, gs2, gs3)
    ssems = (ss0, ss1)
    cid = lax.axis_index("c")
    sid = lax.axis_index("s")
    wid = sid * 2 + cid
    pltpu.sync_copy(idxt_hbm.at[:, pl.ds(wid * BB, BB)], idx_v)

    lane = jnp.arange(LANES, dtype=jnp.int32)

    def gather_cp(t, buf):
        return pltpu.make_async_copy(
            w_hbm.at[idx_v.at[t]], rgs[buf], gsems[buf])

    def store_cps(t, sb):
        cps = []
        for ct in range(MODEL_DIM // 8):
            dst = out_hbm.at[t, ct, wid]
            src = ovs[sb].at[pl.ds(ct * 8, 8)]
            cps.append(pltpu.make_async_copy(src, dst, ssems[sb]))
        return cps

    for buf in range(NG):
        gather_cp(buf, buf).start()

    def group_body(g, carry):
        for buf in range(NG):
            t = g * NG + buf
            sb = buf % NS
            if buf < NS:
                @pl.when(g > 0)
                def _wait_stores():
                    for cp in store_cps(0, sb):
                        cp.wait()
            else:
                for cp in store_cps(0, sb):
                    cp.wait()
            gather_cp(t, buf).wait()

            rg = rgs[buf]
            ov = ovs[sb]

            @plsc.parallel_loop(0, MODEL_DIM, unroll=2)
            def _tr_body(c):
                cols = jnp.full((LANES,), c, dtype=jnp.int32)
                for bg in range(BB // LANES):
                    rows = lane + bg * LANES
                    v = plsc.load_gather(rg, [rows, cols])
                    ov[c, pl.ds(bg * LANES, LANES)] = v * SCALE

            for cp in store_cps(t, sb):
                cp.start()
            @pl.when(t + NG < seq)
            def _refill():
                gather_cp(t + NG, buf).start()
        return carry

    lax.fori_loop(0, seq // NG, group_body, 0)
    for k in range(NS):
        sb = (seq - NS + k) % NS
        for cp in store_cps(0, sb):
            cp.wait()


def kernel(input_ids, weight):
    n_rows, seq = input_ids.shape
    assert n_rows == NUM_WORKERS * BB
    assert seq % NG == 0
    n_ct = MODEL_DIM // 8
    n_bt = n_rows // 128

    idxt = jnp.transpose(input_ids.astype(jnp.int32))

    mesh = plsc.VectorSubcoreMesh(core_axis_name="c", subcore_axis_name="s")
    emb = functools.partial(
        pl.kernel,
        mesh=mesh,
        out_type=jax.ShapeDtypeStruct((seq, n_ct, n_bt, 8, 128), jnp.float32),
        scratch_types=[
            pltpu.VMEM((seq, BB), jnp.int32),
        ] + [pltpu.VMEM((BB, MODEL_DIM), jnp.float32)] * NG
          + [pltpu.VMEM((MODEL_DIM, BB), jnp.float32)] * NS
          + [pltpu.SemaphoreType.DMA] * (NG + NS),
        compiler_params=pltpu.CompilerParams(
            use_tc_tiling_on_sc=False, needs_layout_passes=False),
    )(functools.partial(_emb_kernel_body, seq))

    out5 = emb(idxt, weight)
    return out5.transpose(2, 4, 0, 1, 3).reshape(n_rows, seq, MODEL_DIM)


# speedup vs baseline: 1.6794x; 1.6794x over previous
import functools
import math

import jax
import jax.numpy as jnp
from jax import lax
from jax.experimental import pallas as pl
from jax.experimental.pallas import tpu as pltpu
from jax.experimental.pallas import tpu_sc as plsc

MODEL_DIM = 64
SCALE = math.sqrt(MODEL_DIM)
LANES = 16
NUM_WORKERS = 32
BB = 128
NG = 4
NS = 2


def _emb_kernel_body(seq, idxt_hbm, w_hbm, out_hbm, idx_v, rg0, rg1, rg2, rg3,
                     ov0, ov1, gs0, gs1, gs2, gs3, ss0, ss1):
    rgs = (rg0, rg1, rg2, rg3)
    ovs = (ov0, ov1)
    gsems = (gs0, gs1, gs2, gs3)
    ssems = (ss0, ss1)
    cid = lax.axis_index("c")
    sid = lax.axis_index("s")
    wid = sid * 2 + cid
    pltpu.sync_copy(idxt_hbm.at[:, pl.ds(wid * BB, BB)], idx_v)

    lane = jnp.arange(LANES, dtype=jnp.int32)

    def gather_cp(t, buf):
        return pltpu.make_async_copy(
            w_hbm.at[idx_v.at[t]], rgs[buf], gsems[buf])

    def store_cps(t, sb):
        cps = []
        for ct in range(MODEL_DIM // 8):
            dst = out_hbm.at[t, ct, wid]
            src = ovs[sb].at[pl.ds(ct * 8, 8), pl.ds(0, BB)]
            cps.append(pltpu.make_async_copy(src, dst, ssems[sb]))
        return cps

    for buf in range(NG):
        gather_cp(buf, buf).start()

    def group_body(g, carry):
        for buf in range(NG):
            t = g * NG + buf
            sb = buf % NS
            if buf < NS:
                @pl.when(g > 0)
                def _wait_stores():
                    for cp in store_cps(0, sb):
                        cp.wait()
            else:
                for cp in store_cps(0, sb):
                    cp.wait()
            gather_cp(t, buf).wait()

            rg = rgs[buf]
            ov = ovs[sb]

            @plsc.parallel_loop(0, BB, unroll=2)
            def _tr_body(r):
                rows = jnp.full((LANES,), r, dtype=jnp.int32)
                for cb in range(MODEL_DIM // LANES):
                    v = rg[r, pl.ds(cb * LANES, LANES)]
                    plsc.store_scatter(ov, [lane + cb * LANES, rows],
                                       v * SCALE)

            for cp in store_cps(t, sb):
                cp.start()
            @pl.when(t + NG < seq)
            def _refill():
                gather_cp(t + NG, buf).start()
        return carry

    lax.fori_loop(0, seq // NG, group_body, 0)
    for k in range(NS):
        sb = (seq - NS + k) % NS
        for cp in store_cps(0, sb):
            cp.wait()


def kernel(input_ids, weight):
    n_rows, seq = input_ids.shape
    assert n_rows == NUM_WORKERS * BB
    assert seq % NG == 0
    n_ct = MODEL_DIM // 8
    n_bt = n_rows // 128

    idxt = jnp.transpose(input_ids.astype(jnp.int32))
    w_flat = lax.optimization_barrier(weight.reshape(-1))
    w_lin = w_flat.reshape(weight.shape)

    mesh = plsc.VectorSubcoreMesh(core_axis_name="c", subcore_axis_name="s")
    emb = functools.partial(
        pl.kernel,
        mesh=mesh,
        out_type=jax.ShapeDtypeStruct((seq, n_ct, n_bt, 8, 128), jnp.float32),
        scratch_types=[
            pltpu.VMEM((seq, BB), jnp.int32),
        ] + [pltpu.VMEM((BB, MODEL_DIM), jnp.float32)] * NG
          + [pltpu.VMEM((MODEL_DIM, BB + 1), jnp.float32)] * NS
          + [pltpu.SemaphoreType.DMA] * (NG + NS),
        compiler_params=pltpu.CompilerParams(
            use_tc_tiling_on_sc=False, needs_layout_passes=False),
    )(functools.partial(_emb_kernel_body, seq))

    out5 = emb(idxt, w_lin)
    return out5.transpose(2, 4, 0, 1, 3).reshape(n_rows, seq, MODEL_DIM)
